# baseline (speedup 1.0000x reference)
import jax
import jax.numpy as jnp
from jax import lax
from jax.experimental import pallas as pl
from jax.experimental.pallas import tpu as pltpu
from jax.experimental.pallas import tpu_sc as plsc

B = 16384
DIM = 16
NROWS = 1000000
NC = 2
NS = 16
NW = NC * NS
BPW = B // NW
CHUNK = 128
NCH = BPW // CHUNK
FLAT = DIM * NROWS
TPB = 8000512
C_D = [(d // 8) * TPB + (d % 8) * 128 for d in range(DIM)]


def _sc_body(uid_h, pid_h, nid_h, uembT_h, iembT_h, ubias_h, ibias_h, gb_h,
             pos_h, neg_h,
             uid_v, pid_v, nid_v,
             u_v, p_v, n_v, ub_v, pb_v, nb_v, gb_v, pos_v, neg_v, sem):
  wid = lax.axis_index("s") * NC + lax.axis_index("c")
  base = wid * BPW

  pltpu.sync_copy(gb_h, gb_v)

  for j in range(NCH):
    sl_h = pl.ds(base + j * CHUNK, CHUNK)
    pltpu.sync_copy(uid_h.at[sl_h], uid_v.at[j])
    pltpu.sync_copy(pid_h.at[sl_h], pid_v.at[j])
    pltpu.sync_copy(nid_h.at[sl_h], nid_v.at[j])


  copies = []
  for j in range(NCH):
    sl = pl.ds(j * CHUNK, CHUNK)
    copies.append(pltpu.make_async_copy(
        ubias_h.at[uid_v.at[j]], ub_v.at[sl], sem))
    copies.append(pltpu.make_async_copy(
        ibias_h.at[pid_v.at[j]], pb_v.at[sl], sem))
    copies.append(pltpu.make_async_copy(
        ibias_h.at[nid_v.at[j]], nb_v.at[sl], sem))
    for d in range(DIM):
      copies.append(pltpu.make_async_copy(
          uembT_h.at[d].at[uid_v.at[j]], u_v.at[d, sl], sem))
      copies.append(pltpu.make_async_copy(
          iembT_h.at[d].at[pid_v.at[j]], p_v.at[d, sl], sem))
      copies.append(pltpu.make_async_copy(
          iembT_h.at[d].at[nid_v.at[j]], n_v.at[d, sl], sem))
  for c in copies:
    c.start()
  for c in copies:
    c.wait()

  gbv = gb_v[...]

  def group(s, carry):
    sl = pl.ds(s * 16, 16)
    ubv = ub_v[sl]
    pos = gbv + ubv + pb_v[sl]
    neg = gbv + ubv + nb_v[sl]
    for d in range(DIM):
      ud = u_v[d, sl]
      pos = pos + ud * p_v[d, sl]
      neg = neg + ud * n_v[d, sl]
    pos_v[sl] = pos
    neg_v[sl] = neg
    return carry

  lax.fori_loop(0, BPW // 16, group, 0)

  pltpu.sync_copy(pos_v, pos_h.at[pl.ds(base, BPW)])
  pltpu.sync_copy(neg_v, neg_h.at[pl.ds(base, BPW)])


def kernel(user_ids, pos_item_ids, neg_item_ids, user_emb_w, item_emb_w,
           user_bias_w, item_bias_w, global_bias):
  gb16 = jnp.broadcast_to(global_bias.astype(jnp.float32), (16,))
  uembT = user_emb_w.T
  iembT = item_emb_w.T
  ubias_flat = user_bias_w.reshape(-1)
  ibias_flat = item_bias_w.reshape(-1)
  mesh = plsc.VectorSubcoreMesh(core_axis_name="c", subcore_axis_name="s",
                                num_cores=NC, num_subcores=NS)
  f = pl.kernel(
      _sc_body,
      out_type=(jax.ShapeDtypeStruct((B,), jnp.float32),
                jax.ShapeDtypeStruct((B,), jnp.float32)),
      mesh=mesh,
      compiler_params=pltpu.CompilerParams(needs_layout_passes=False,
                                           use_tc_tiling_on_sc=False),
      scratch_types=[
          pltpu.VMEM((NCH, CHUNK), jnp.int32),
          pltpu.VMEM((NCH, CHUNK), jnp.int32),
          pltpu.VMEM((NCH, CHUNK), jnp.int32),
          pltpu.VMEM((DIM, BPW), jnp.float32),
          pltpu.VMEM((DIM, BPW), jnp.float32),
          pltpu.VMEM((DIM, BPW), jnp.float32),
          pltpu.VMEM((BPW,), jnp.float32),
          pltpu.VMEM((BPW,), jnp.float32),
          pltpu.VMEM((BPW,), jnp.float32),
          pltpu.VMEM((16,), jnp.float32),
          pltpu.VMEM((BPW,), jnp.float32),
          pltpu.VMEM((BPW,), jnp.float32),
          pltpu.SemaphoreType.DMA,
      ],
  )
  return f(user_ids, pos_item_ids, neg_item_ids, uembT, iembT,
           ubias_flat, ibias_flat, gb16)


# speedup vs baseline: 3.1728x; 3.1728x over previous
import jax
import jax.numpy as jnp
from jax import lax
from jax.experimental import pallas as pl
from jax.experimental.pallas import tpu as pltpu
from jax.experimental.pallas import tpu_sc as plsc

B = 16384
DIM = 16
NROWS = 1000000
FLAT = NROWS * DIM
NC = 2
NS = 16
NW = NC * NS
BPW = B // NW
CHUNK = 128
NCH = BPW // CHUNK


def _sc_body(uid_h, pid_h, nid_h, uflat_h, iflat_h, ubias_h, ibias_h, gb_h,
             pos_h, neg_h,
             uid_v, pid_v, nid_v, ubase_v, pbase_v, nbase_v,
             u_v, p_v, n_v, ub_v, pb_v, nb_v, gb_v, pos_v, neg_v, sem):
  wid = lax.axis_index("s") * NC + lax.axis_index("c")
  base = wid * BPW

  pltpu.sync_copy(gb_h, gb_v)

  for j in range(NCH):
    sl_h = pl.ds(base + j * CHUNK, CHUNK)
    pltpu.sync_copy(uid_h.at[sl_h], uid_v.at[j])
    pltpu.sync_copy(pid_h.at[sl_h], pid_v.at[j])
    pltpu.sync_copy(nid_h.at[sl_h], nid_v.at[j])

  for j in range(NCH):
    def mkbase(s, carry):
      sl = pl.ds(s * 16, 16)
      for ids, bases in ((uid_v, ubase_v), (pid_v, pbase_v), (nid_v, nbase_v)):
        b0 = ids[j, sl] * DIM
        for r in range(8):
          bases[r, j, sl] = b0 + r
      return carry
    lax.fori_loop(0, CHUNK // 16, mkbase, 0)

  copies = []
  for j in range(NCH):
    sl = pl.ds(j * CHUNK, CHUNK)
    copies.append(pltpu.make_async_copy(
        ubias_h.at[uid_v.at[j]], ub_v.at[sl], sem))
    copies.append(pltpu.make_async_copy(
        ibias_h.at[pid_v.at[j]], pb_v.at[sl], sem))
    copies.append(pltpu.make_async_copy(
        ibias_h.at[nid_v.at[j]], nb_v.at[sl], sem))
    for d in range(DIM):
      r, off = d % 8, (d // 8) * 8
      usrc = uflat_h.at[pl.ds(off, FLAT - off)]
      isrc = iflat_h.at[pl.ds(off, FLAT - off)]
      copies.append(pltpu.make_async_copy(
          usrc.at[ubase_v.at[r, j]], u_v.at[d, sl], sem))
      copies.append(pltpu.make_async_copy(
          isrc.at[pbase_v.at[r, j]], p_v.at[d, sl], sem))
      copies.append(pltpu.make_async_copy(
          isrc.at[nbase_v.at[r, j]], n_v.at[d, sl], sem))
  for c in copies:
    c.start()
  for c in copies:
    c.wait()

  gbv = gb_v[...]

  def group(s, carry):
    sl = pl.ds(s * 16, 16)
    ubv = ub_v[sl]
    pos = gbv + ubv + pb_v[sl]
    neg = gbv + ubv + nb_v[sl]
    for d in range(DIM):
      ud = u_v[d, sl]
      pos = pos + ud * p_v[d, sl]
      neg = neg + ud * n_v[d, sl]
    pos_v[sl] = pos
    neg_v[sl] = neg
    return carry

  lax.fori_loop(0, BPW // 16, group, 0)

  pltpu.sync_copy(pos_v, pos_h.at[pl.ds(base, BPW)])
  pltpu.sync_copy(neg_v, neg_h.at[pl.ds(base, BPW)])


def kernel(user_ids, pos_item_ids, neg_item_ids, user_emb_w, item_emb_w,
           user_bias_w, item_bias_w, global_bias):
  gb16 = jnp.broadcast_to(global_bias.astype(jnp.float32), (16,))
  uflat = user_emb_w.reshape(-1)
  iflat = item_emb_w.reshape(-1)
  ubias_flat = user_bias_w.reshape(-1)
  ibias_flat = item_bias_w.reshape(-1)
  mesh = plsc.VectorSubcoreMesh(core_axis_name="c", subcore_axis_name="s",
                                num_cores=NC, num_subcores=NS)
  f = pl.kernel(
      _sc_body,
      out_type=(jax.ShapeDtypeStruct((B,), jnp.float32),
                jax.ShapeDtypeStruct((B,), jnp.float32)),
      mesh=mesh,
      compiler_params=pltpu.CompilerParams(needs_layout_passes=False),
      scratch_types=[
          pltpu.VMEM((NCH, CHUNK), jnp.int32),
          pltpu.VMEM((NCH, CHUNK), jnp.int32),
          pltpu.VMEM((NCH, CHUNK), jnp.int32),
          pltpu.VMEM((8, NCH, CHUNK), jnp.int32),
          pltpu.VMEM((8, NCH, CHUNK), jnp.int32),
          pltpu.VMEM((8, NCH, CHUNK), jnp.int32),
          pltpu.VMEM((DIM, BPW), jnp.float32),
          pltpu.VMEM((DIM, BPW), jnp.float32),
          pltpu.VMEM((DIM, BPW), jnp.float32),
          pltpu.VMEM((BPW,), jnp.float32),
          pltpu.VMEM((BPW,), jnp.float32),
          pltpu.VMEM((BPW,), jnp.float32),
          pltpu.VMEM((16,), jnp.float32),
          pltpu.VMEM((BPW,), jnp.float32),
          pltpu.VMEM((BPW,), jnp.float32),
          pltpu.SemaphoreType.DMA,
      ],
  )
  return f(user_ids, pos_item_ids, neg_item_ids, uflat, iflat,
           ubias_flat, ibias_flat, gb16)
